# Initial kernel scaffold; baseline (speedup 1.0000x reference)
import jax
import jax.numpy as jnp
from jax.experimental import pallas as pl


def kernel(input_ids, word_embeddings, ln_weight, ln_bias):
    raise NotImplementedError("write your pallas kernel here")


# speedup vs baseline: 2.2559x; 2.2559x over previous
import jax
import jax.numpy as jnp
from jax import lax
from jax.experimental import pallas as pl
from jax.experimental.pallas import tpu as pltpu
from jax.experimental.pallas import tpu_sc as plsc

D = 128
EPS = 1e-12
NW = 32
CHUNK = 128
NV = D // 16


def _ln_body(ids_hbm, table_hbm, w_hbm, b_hbm, out_hbm,
             idx_v, rows_v, w_v, b_v, sem):
    npw = ids_hbm.shape[0] // NW
    ncw = npw // CHUNK
    wid = lax.axis_index("s") * 2 + lax.axis_index("c")
    rbase = pl.multiple_of(wid * npw, CHUNK)
    pltpu.sync_copy(ids_hbm.at[pl.ds(rbase, npw)], idx_v)
    pltpu.sync_copy(w_hbm, w_v)
    pltpu.sync_copy(b_hbm, b_v)
    wv = [w_v[pl.ds(16 * i, 16)] for i in range(NV)]
    bv = [b_v[pl.ds(16 * i, 16)] for i in range(NV)]

    lanes = lax.iota(jnp.int32, 16)
    perms = [(lanes ^ sh)[:, None] for sh in (1, 2, 4, 8)]
    dnums = lax.GatherDimensionNumbers(
        offset_dims=(), collapsed_slice_dims=(0,), start_index_map=(0,))

    def allsum(v):
        for p in perms:
            v = v + lax.gather(v, p, dnums, (1,),
                               mode=lax.GatherScatterMode.PROMISE_IN_BOUNDS)
        return v

    def chunk_body(j, carry):
        coff = pl.multiple_of(j * CHUNK, CHUNK)
        idx_c = idx_v.at[pl.ds(coff, CHUNK)]
        pltpu.async_copy(table_hbm.at[idx_c], rows_v, sem).wait()

        def row_body(r, c2):
            vs = [rows_v[r, pl.ds(16 * i, 16)] for i in range(NV)]
            s1 = vs[0]
            s2 = vs[0] * vs[0]
            for v in vs[1:]:
                s1 = s1 + v
                s2 = s2 + v * v
            s1b = allsum(s1)
            s2b = allsum(s2)
            mean = s1b * (1.0 / D)
            var = jnp.maximum(s2b * (1.0 / D) - mean * mean, 0.0) + EPS
            yi = jnp.int32(0x5F3759DF) - (lax.bitcast_convert_type(var, jnp.int32) >> 1)
            y = lax.bitcast_convert_type(yi, jnp.float32)
            h = var * 0.5
            for _ in range(3):
                y = y * (1.5 - h * y * y)
            for i in range(NV):
                rows_v[r, pl.ds(16 * i, 16)] = (vs[i] - mean) * y * wv[i] + bv[i]
            return c2

        lax.fori_loop(0, CHUNK, row_body, 0)
        ooff = pl.multiple_of(rbase + coff, CHUNK)
        pltpu.sync_copy(rows_v, out_hbm.at[pl.ds(ooff, CHUNK)])
        return carry

    lax.fori_loop(0, ncw, chunk_body, 0)


def kernel(input_ids, word_embeddings, ln_weight, ln_bias):
    B, T = input_ids.shape
    V, Dd = word_embeddings.shape
    N = B * T
    ids_flat = input_ids.reshape(N)
    mesh = plsc.VectorSubcoreMesh(core_axis_name="c", subcore_axis_name="s")
    f = pl.kernel(
        _ln_body,
        mesh=mesh,
        out_type=jax.ShapeDtypeStruct((N, Dd), jnp.float32),
        scratch_types=[
            pltpu.VMEM((N // NW,), jnp.int32),
            pltpu.VMEM((CHUNK, Dd), jnp.float32),
            pltpu.VMEM((Dd,), jnp.float32),
            pltpu.VMEM((Dd,), jnp.float32),
            pltpu.SemaphoreType.DMA,
        ],
    )
    out = f(ids_flat, word_embeddings, ln_weight, ln_bias)
    return out.reshape(B, T, Dd)


# speedup vs baseline: 4.4640x; 1.9788x over previous
import jax
import jax.numpy as jnp
from jax import lax
from jax.experimental import pallas as pl
from jax.experimental.pallas import tpu as pltpu
from jax.experimental.pallas import tpu_sc as plsc

D = 128
EPS = 1e-12
NW = 32
CHUNK = 128
NV = D // 16


def _ln_body(ids_hbm, table_hbm, w_hbm, b_hbm, out_hbm,
             idx_v, rows_a, rows_b, w_v, b_v, sga, sgb, ssa, ssb):
    npw = ids_hbm.shape[0] // NW
    ncw = npw // CHUNK
    npairs = ncw // 2
    wid = lax.axis_index("s") * 2 + lax.axis_index("c")
    rbase = pl.multiple_of(wid * npw, CHUNK)
    pltpu.sync_copy(ids_hbm.at[pl.ds(rbase, npw)], idx_v)
    pltpu.sync_copy(w_hbm, w_v)
    pltpu.sync_copy(b_hbm, b_v)
    wv = [w_v[pl.ds(16 * i, 16)] for i in range(NV)]
    bv = [b_v[pl.ds(16 * i, 16)] for i in range(NV)]

    lanes = lax.iota(jnp.int32, 16)
    perms = [(lanes ^ sh)[:, None] for sh in (1, 2, 4, 8)]
    dnums = lax.GatherDimensionNumbers(
        offset_dims=(), collapsed_slice_dims=(0,), start_index_map=(0,))

    def allsum(v):
        for p in perms:
            v = v + lax.gather(v, p, dnums, (1,),
                               mode=lax.GatherScatterMode.PROMISE_IN_BOUNDS)
        return v

    def ln_row(buf, r):
        vs = [buf[r, pl.ds(16 * i, 16)] for i in range(NV)]
        s1 = vs[0]
        s2 = vs[0] * vs[0]
        for v in vs[1:]:
            s1 = s1 + v
            s2 = s2 + v * v
        s1b = allsum(s1)
        s2b = allsum(s2)
        mean = s1b * (1.0 / D)
        var = jnp.maximum(s2b * (1.0 / D) - mean * mean, 0.0) + EPS
        yi = jnp.int32(0x5F3759DF) - (lax.bitcast_convert_type(var, jnp.int32) >> 1)
        y = lax.bitcast_convert_type(yi, jnp.float32)
        h = var * 0.5
        for _ in range(2):
            y = y * (1.5 - h * y * y)
        my = mean * y
        for i in range(NV):
            buf[r, pl.ds(16 * i, 16)] = (vs[i] * y - my) * wv[i] + bv[i]

    def compute(buf):
        def rb(r, c):
            ln_row(buf, 2 * r)
            ln_row(buf, 2 * r + 1)
            return c
        lax.fori_loop(0, CHUNK // 2, rb, 0)

    def start_gather(j, buf, sem):
        coff = pl.multiple_of(j * CHUNK, CHUNK)
        pltpu.async_copy(table_hbm.at[idx_v.at[pl.ds(coff, CHUNK)]], buf, sem)

    def wait_gather(buf, sem):
        pltpu.make_async_copy(table_hbm.at[pl.ds(0, CHUNK)], buf, sem).wait()

    def start_store(j, buf, sem):
        ooff = pl.multiple_of(rbase + j * CHUNK, CHUNK)
        pltpu.async_copy(buf, out_hbm.at[pl.ds(ooff, CHUNK)], sem)

    def wait_store(buf, sem):
        pltpu.make_async_copy(buf, out_hbm.at[pl.ds(rbase, CHUNK)], sem).wait()

    start_gather(0, rows_a, sga)
    start_gather(1, rows_b, sgb)

    def pair(k, c):
        j0 = 2 * k
        wait_gather(rows_a, sga)
        compute(rows_a)
        start_store(j0, rows_a, ssa)
        wait_gather(rows_b, sgb)
        compute(rows_b)
        start_store(j0 + 1, rows_b, ssb)

        @pl.when(k < npairs - 1)
        def _prefetch():
            wait_store(rows_a, ssa)
            start_gather(j0 + 2, rows_a, sga)
            wait_store(rows_b, ssb)
            start_gather(j0 + 3, rows_b, sgb)

        return c

    lax.fori_loop(0, npairs, pair, 0)
    wait_store(rows_a, ssa)
    wait_store(rows_b, ssb)


def kernel(input_ids, word_embeddings, ln_weight, ln_bias):
    B, T = input_ids.shape
    V, Dd = word_embeddings.shape
    N = B * T
    ids_flat = input_ids.reshape(N)
    mesh = plsc.VectorSubcoreMesh(core_axis_name="c", subcore_axis_name="s")
    f = pl.kernel(
        _ln_body,
        mesh=mesh,
        out_type=jax.ShapeDtypeStruct((N, Dd), jnp.float32),
        scratch_types=[
            pltpu.VMEM((N // NW,), jnp.int32),
            pltpu.VMEM((CHUNK, Dd), jnp.float32),
            pltpu.VMEM((CHUNK, Dd), jnp.float32),
            pltpu.VMEM((Dd,), jnp.float32),
            pltpu.VMEM((Dd,), jnp.float32),
            pltpu.SemaphoreType.DMA,
            pltpu.SemaphoreType.DMA,
            pltpu.SemaphoreType.DMA,
            pltpu.SemaphoreType.DMA,
        ],
    )
    out = f(ids_flat, word_embeddings, ln_weight, ln_bias)
    return out.reshape(B, T, Dd)


# speedup vs baseline: 5.8156x; 1.3028x over previous
import jax
import jax.numpy as jnp
from jax import lax
from jax.experimental import pallas as pl
from jax.experimental.pallas import tpu as pltpu
from jax.experimental.pallas import tpu_sc as plsc

D = 128
EPS = 1e-12
NW = 32
CHUNK = 128
LN_BLK = 2000


def _ln_table_body(x_ref, w_ref, b_ref, o_ref):
    x = x_ref[...]
    mean = jnp.mean(x, axis=1, keepdims=True)
    xc = x - mean
    var = jnp.mean(xc * xc, axis=1, keepdims=True)
    o_ref[...] = xc * lax.rsqrt(var + EPS) * w_ref[...] + b_ref[...]


def _gather_body(ids_hbm, table_hbm, out_hbm, idx_v, rows_a, rows_b,
                 sga, sgb, ssa, ssb):
    npw = ids_hbm.shape[0] // NW
    ncw = npw // CHUNK
    npairs = ncw // 2
    wid = lax.axis_index("s") * 2 + lax.axis_index("c")
    rbase = pl.multiple_of(wid * npw, CHUNK)
    pltpu.sync_copy(ids_hbm.at[pl.ds(rbase, npw)], idx_v)

    def start_gather(j, buf, sem):
        coff = pl.multiple_of(j * CHUNK, CHUNK)
        pltpu.async_copy(table_hbm.at[idx_v.at[pl.ds(coff, CHUNK)]], buf, sem)

    def wait_gather(buf, sem):
        pltpu.make_async_copy(table_hbm.at[pl.ds(0, CHUNK)], buf, sem).wait()

    def start_store(j, buf, sem):
        ooff = pl.multiple_of(rbase + j * CHUNK, CHUNK)
        pltpu.async_copy(buf, out_hbm.at[pl.ds(ooff, CHUNK)], sem)

    def wait_store(buf, sem):
        pltpu.make_async_copy(buf, out_hbm.at[pl.ds(rbase, CHUNK)], sem).wait()

    start_gather(0, rows_a, sga)
    start_gather(1, rows_b, sgb)

    def pair(k, c):
        j0 = 2 * k
        wait_gather(rows_a, sga)
        start_store(j0, rows_a, ssa)
        wait_gather(rows_b, sgb)
        start_store(j0 + 1, rows_b, ssb)

        @pl.when(k < npairs - 1)
        def _prefetch():
            wait_store(rows_a, ssa)
            start_gather(j0 + 2, rows_a, sga)
            wait_store(rows_b, ssb)
            start_gather(j0 + 3, rows_b, sgb)

        return c

    lax.fori_loop(0, npairs, pair, 0)
    wait_store(rows_a, ssa)
    wait_store(rows_b, ssb)


def kernel(input_ids, word_embeddings, ln_weight, ln_bias):
    B, T = input_ids.shape
    V, Dd = word_embeddings.shape
    N = B * T

    ln_table = pl.pallas_call(
        _ln_table_body,
        grid=(V // LN_BLK,),
        in_specs=[
            pl.BlockSpec((LN_BLK, Dd), lambda i: (i, 0)),
            pl.BlockSpec((Dd,), lambda i: (0,)),
            pl.BlockSpec((Dd,), lambda i: (0,)),
        ],
        out_specs=pl.BlockSpec((LN_BLK, Dd), lambda i: (i, 0)),
        out_shape=jax.ShapeDtypeStruct((V, Dd), jnp.float32),
    )(word_embeddings, ln_weight, ln_bias)

    ids_flat = input_ids.reshape(N)
    mesh = plsc.VectorSubcoreMesh(core_axis_name="c", subcore_axis_name="s")
    f = pl.kernel(
        _gather_body,
        mesh=mesh,
        out_type=jax.ShapeDtypeStruct((N, Dd), jnp.float32),
        scratch_types=[
            pltpu.VMEM((N // NW,), jnp.int32),
            pltpu.VMEM((CHUNK, Dd), jnp.float32),
            pltpu.VMEM((CHUNK, Dd), jnp.float32),
            pltpu.SemaphoreType.DMA,
            pltpu.SemaphoreType.DMA,
            pltpu.SemaphoreType.DMA,
            pltpu.SemaphoreType.DMA,
        ],
    )
    out = f(ids_flat, ln_table)
    return out.reshape(B, T, Dd)


# speedup vs baseline: 6.0450x; 1.0394x over previous
import jax
import jax.numpy as jnp
from jax import lax
from jax.experimental import pallas as pl
from jax.experimental.pallas import tpu as pltpu
from jax.experimental.pallas import tpu_sc as plsc

D = 128
EPS = 1e-12
NW = 32
CHUNK = 80
NBUF = 4
LN_BLK = 2000


def _ln_table_body(x_ref, w_ref, b_ref, o_ref):
    x = x_ref[...]
    mean = jnp.mean(x, axis=1, keepdims=True)
    xc = x - mean
    var = jnp.mean(xc * xc, axis=1, keepdims=True)
    o_ref[...] = xc * lax.rsqrt(var + EPS) * w_ref[...] + b_ref[...]


def _gather_body(ids_hbm, table_hbm, out_hbm, idx_v, *rest):
    bufs = rest[:NBUF]
    sg = rest[NBUF:2 * NBUF]
    ss = rest[2 * NBUF:3 * NBUF]
    npw = ids_hbm.shape[0] // NW
    ncw = npw // CHUNK
    ngroups = ncw // NBUF
    wid = lax.axis_index("s") * 2 + lax.axis_index("c")
    rbase = pl.multiple_of(wid * npw, CHUNK)
    pltpu.sync_copy(ids_hbm.at[pl.ds(rbase, npw)], idx_v)

    def start_gather(j, buf, sem):
        coff = pl.multiple_of(j * CHUNK, CHUNK)
        pltpu.async_copy(table_hbm.at[idx_v.at[pl.ds(coff, CHUNK)]], buf, sem)

    def wait_gather(buf, sem):
        pltpu.make_async_copy(table_hbm.at[pl.ds(0, CHUNK)], buf, sem).wait()

    def start_store(j, buf, sem):
        ooff = pl.multiple_of(rbase + j * CHUNK, CHUNK)
        pltpu.async_copy(buf, out_hbm.at[pl.ds(ooff, CHUNK)], sem)

    def wait_store(buf, sem):
        pltpu.make_async_copy(buf, out_hbm.at[pl.ds(rbase, CHUNK)], sem).wait()

    for b in range(NBUF):
        start_gather(b, bufs[b], sg[b])

    def group(k, c):
        j0 = NBUF * k
        for b in range(NBUF):
            wait_gather(bufs[b], sg[b])
            start_store(j0 + b, bufs[b], ss[b])

        @pl.when(k < ngroups - 1)
        def _prefetch():
            for b in range(NBUF):
                wait_store(bufs[b], ss[b])
                start_gather(j0 + NBUF + b, bufs[b], sg[b])

        return c

    lax.fori_loop(0, ngroups, group, 0)
    for b in range(NBUF):
        wait_store(bufs[b], ss[b])


def kernel(input_ids, word_embeddings, ln_weight, ln_bias):
    B, T = input_ids.shape
    V, Dd = word_embeddings.shape
    N = B * T

    ln_table = pl.pallas_call(
        _ln_table_body,
        grid=(V // LN_BLK,),
        in_specs=[
            pl.BlockSpec((LN_BLK, Dd), lambda i: (i, 0)),
            pl.BlockSpec((Dd,), lambda i: (0,)),
            pl.BlockSpec((Dd,), lambda i: (0,)),
        ],
        out_specs=pl.BlockSpec((LN_BLK, Dd), lambda i: (i, 0)),
        out_shape=jax.ShapeDtypeStruct((V, Dd), jnp.float32),
    )(word_embeddings, ln_weight, ln_bias)

    ids_flat = input_ids.reshape(N)
    mesh = plsc.VectorSubcoreMesh(core_axis_name="c", subcore_axis_name="s")
    f = pl.kernel(
        _gather_body,
        mesh=mesh,
        out_type=jax.ShapeDtypeStruct((N, Dd), jnp.float32),
        scratch_types=(
            [pltpu.VMEM((N // NW,), jnp.int32)]
            + [pltpu.VMEM((CHUNK, Dd), jnp.float32) for _ in range(NBUF)]
            + [pltpu.SemaphoreType.DMA for _ in range(2 * NBUF)]
        ),
    )
    out = f(ids_flat, ln_table)
    return out.reshape(B, T, Dd)
